# baseline (speedup 1.0000x reference)
import functools

import jax
import jax.numpy as jnp
from jax import lax
from jax.experimental import pallas as pl
from jax.experimental.pallas import tpu as pltpu
from jax.experimental.pallas import tpu_sc as plsc

NC = 2
NS = 16
NW = NC * NS
CH = 128


def _gather_body(idx_hbm, table_hbm, out_hbm, idx_v, buf0, buf1, sem0, sem1,
                 *, n_chunk, emb):
    wid = lax.axis_index("s") * NC + lax.axis_index("c")
    pltpu.sync_copy(idx_hbm.at[wid], idx_v)
    base = wid * (n_chunk * CH)

    bufs = (buf0, buf1)
    sems = (sem0, sem1)

    def start(j):
        return pltpu.async_copy(
            table_hbm.at[idx_v.at[j]], bufs[j % 2], sems[j % 2])

    descs = [None] * n_chunk
    descs[0] = start(0)
    for j in range(n_chunk):
        if j + 1 < n_chunk:
            descs[j + 1] = start(j + 1)
        descs[j].wait()
        pltpu.sync_copy(bufs[j % 2], out_hbm.at[pl.ds(base + j * CH, CH)])


def _sc_gather(idx, table, n_rows, emb):
    n_chunk = n_rows // (NW * CH)
    idx3 = idx.reshape(NW, n_chunk, CH)
    mesh = plsc.VectorSubcoreMesh(core_axis_name="c", subcore_axis_name="s")
    body = functools.partial(_gather_body, n_chunk=n_chunk, emb=emb)
    return pl.kernel(
        body,
        out_type=jax.ShapeDtypeStruct((n_rows, emb), jnp.float32),
        mesh=mesh,
        scratch_types=[
            pltpu.VMEM((n_chunk, CH), jnp.int32),
            pltpu.VMEM((CH, emb), jnp.float32),
            pltpu.VMEM((CH, emb), jnp.float32),
            pltpu.SemaphoreType.DMA,
            pltpu.SemaphoreType.DMA,
        ],
        compiler_params=pltpu.CompilerParams(use_tc_tiling_on_sc=False),
    )(idx3, table)


def _mlp_body(flat_ref, w1_ref, b1_ref, w2_ref, b2_ref, out_ref):
    h = jnp.maximum(
        jnp.dot(flat_ref[...], w1_ref[...],
                preferred_element_type=jnp.float32) + b1_ref[...], 0.0)
    logits = jnp.dot(h, w2_ref[...],
                     preferred_element_type=jnp.float32) + b2_ref[...]
    m = jnp.max(logits, axis=1, keepdims=True)
    e = logits - m
    lse = jnp.log(jnp.sum(jnp.exp(e), axis=1, keepdims=True))
    out_ref[...] = e - lse


def _tc_mlp(flat, w1, b1, w2, b2, num_tags):
    bs, in_dim = flat.shape
    hidden = w1.shape[1]
    blk = 2048
    grid = bs // blk
    return pl.pallas_call(
        _mlp_body,
        grid=(grid,),
        in_specs=[
            pl.BlockSpec((blk, in_dim), lambda i: (i, 0)),
            pl.BlockSpec((in_dim, hidden), lambda i: (0, 0)),
            pl.BlockSpec((1, hidden), lambda i: (0, 0)),
            pl.BlockSpec((hidden, num_tags), lambda i: (0, 0)),
            pl.BlockSpec((1, num_tags), lambda i: (0, 0)),
        ],
        out_specs=pl.BlockSpec((blk, num_tags), lambda i: (i, 0)),
        out_shape=jax.ShapeDtypeStruct((bs, num_tags), jnp.float32),
    )(flat, w1, b1.reshape(1, hidden), w2, b2.reshape(1, num_tags))


def kernel(Xtoks_IDs, emb_table, W1, b1, W2, b2):
    bs, seq = Xtoks_IDs.shape
    emb = emb_table.shape[1]
    num_tags = W2.shape[1]
    idx = Xtoks_IDs.astype(jnp.int32).reshape(-1)
    rows = _sc_gather(idx, emb_table, bs * seq, emb)
    flat = rows.reshape(bs, seq * emb)
    return _tc_mlp(flat, W1, b1, W2, b2, num_tags)
